# baseline (speedup 1.0000x reference)
import functools

import jax
import jax.numpy as jnp
from jax import lax
from jax.experimental import pallas as pl
from jax.experimental.pallas import tpu as pltpu
from jax.experimental.pallas import tpu_sc as plsc

NUM_ENTITIES = 1000000
D = 64
DP = 128
B = 4096
F = 100
TOT = B * F

NC, NS = 2, 16
NW = NC * NS
PER_W = TOT // NW
C = 128
NCHUNK = PER_W // C
NG = NCHUNK
NPAIR = NG // 2


@functools.partial(
    pl.kernel,
    out_type=(
        jax.ShapeDtypeStruct((TOT, DP), jnp.float32),
        jax.ShapeDtypeStruct((TOT, DP), jnp.float32),
    ),
    mesh=plsc.VectorSubcoreMesh(core_axis_name="c", subcore_axis_name="s"),
    compiler_params=pltpu.CompilerParams(use_tc_tiling_on_sc=True),
    scratch_types=[
        pltpu.VMEM((NCHUNK, C), jnp.int32),
        pltpu.VMEM((2, C, DP), jnp.float32),
        pltpu.VMEM((2, C, DP), jnp.float32),
        pltpu.SemaphoreType.DMA,
        pltpu.SemaphoreType.DMA,
        pltpu.SemaphoreType.DMA,
        pltpu.SemaphoreType.DMA,
    ],
)
def _bracket_gather(idx_hbm, bra_hbm, ket_hbm, bra_out, ket_out,
                    idx_v, bra_v, ket_v, gsem0, gsem1, ssem0, ssem1):
    wid = lax.axis_index("s") * NC + lax.axis_index("c")
    base = wid * PER_W
    gsem = (gsem0, gsem1)
    ssem = (ssem0, ssem1)

    pltpu.sync_copy(idx_hbm.at[wid], idx_v)

    def fire_gathers(g, set_):
        pltpu.async_copy(bra_hbm.at[idx_v.at[g]], bra_v.at[set_], gsem[set_])
        pltpu.async_copy(ket_hbm.at[idx_v.at[g]], ket_v.at[set_], gsem[set_])

    def wait_gathers(set_):
        pltpu.make_async_copy(
            bra_out.at[pl.ds(0, C)], bra_v.at[set_], gsem[set_]).wait()
        pltpu.make_async_copy(
            ket_out.at[pl.ds(0, C)], ket_v.at[set_], gsem[set_]).wait()

    def fire_stores(g, set_):
        off = base + g * C
        pltpu.async_copy(bra_v.at[set_], bra_out.at[pl.ds(off, C)],
                         ssem[set_])
        pltpu.async_copy(ket_v.at[set_], ket_out.at[pl.ds(off, C)],
                         ssem[set_])

    def wait_stores(set_):
        pltpu.make_async_copy(
            bra_v.at[set_], bra_out.at[pl.ds(0, C)], ssem[set_]).wait()
        pltpu.make_async_copy(
            ket_v.at[set_], ket_out.at[pl.ds(0, C)], ssem[set_]).wait()

    fire_gathers(0, 0)

    def pair(p, carry):
        for parity in range(2):
            g = p * 2 + parity
            other = 1 - parity
            @pl.when(g >= 1)
            def _():
                wait_stores(other)
            @pl.when(g + 1 < NG)
            def _():
                fire_gathers(g + 1, other)
            wait_gathers(parity)
            fire_stores(g, parity)
        return carry

    lax.fori_loop(0, NPAIR, pair, 0)
    wait_stores((NG - 1) % 2)


def kernel(index, bra_weight, ket_weight):
    idx = index.reshape(NW, NCHUNK, C).astype(jnp.int32)
    bra_p = jnp.pad(bra_weight, ((0, 0), (0, DP - D)))
    ket_p = jnp.pad(ket_weight, ((0, 0), (0, DP - D)))
    bra_flat, ket_flat = _bracket_gather(idx, bra_p, ket_p)
    return (bra_flat[:, :D].reshape(B, F, D),
            ket_flat[:, :D].reshape(B, F, D))
